# baseline (speedup 1.0000x reference)
import jax
import jax.numpy as jnp
from jax.experimental import pallas as pl
from jax.experimental.pallas import tpu as pltpu

_N = 10000
_H = 2048
_D = 128

_K_BLK = 512
_NP = 10240
_NK = _NP // _K_BLK
_SRC_BLK = 400
_NS = _N // _SRC_BLK
_OUT_BLK = 2000
_NO = _N // _OUT_BLK


def _mega_kernel(hgt_hbm, hgs_hbm, x0_hbm, att_ref, out_hbm,
                 hgs16, x116, t32, sem):
    x116[pl.ds(_N, _NP - _N), :] = jnp.zeros((_NP - _N, _D), jnp.bfloat16)

    a = att_ref[0, :]
    e = jnp.exp(a - jnp.max(a))
    w = e / jnp.sum(e)
    w0, w1, w2 = w[0], w[1], w[2]

    def load_x0(x0_blk):
        i = pl.program_id(0)
        x116[pl.ds(i * _OUT_BLK, _OUT_BLK), :] = x0_blk[...].astype(jnp.bfloat16)

    pltpu.emit_pipeline(
        load_x0, grid=(_NO,),
        in_specs=[pl.BlockSpec((_OUT_BLK, _D), lambda i: (i, 0))],
    )(x0_hbm)

    def tar_phase(hgt_blk):
        k = pl.program_id(0)
        blk = hgt_blk[...]
        lane = jax.lax.broadcasted_iota(jnp.int32, (_H, _K_BLK), 1)
        blk = jnp.where(k * _K_BLK + lane < _N, blk, 0.0)
        xk = x116[pl.ds(k * _K_BLK, _K_BLK), :]
        r = jnp.dot(blk.astype(jnp.bfloat16), xk,
                    preferred_element_type=jnp.float32)

        @pl.when(k == 0)
        def _init():
            t32[...] = r

        @pl.when(k > 0)
        def _accum():
            t32[...] += r

    tar_pipeline = pltpu.emit_pipeline(
        tar_phase, grid=(_NK,),
        in_specs=[pl.BlockSpec((_H, _K_BLK), lambda k: (0, k))],
    )

    tar_pipeline(hgt_hbm)

    def p2_src1(hgs_blk, x0_blk):
        i = pl.program_id(0)
        rows = pl.ds(i * _SRC_BLK, _SRC_BLK)
        blk = hgs_blk[...].astype(jnp.bfloat16)
        hgs16[rows, :] = blk
        s = jnp.dot(blk, t32[...].astype(jnp.bfloat16),
                    preferred_element_type=jnp.float32)
        x1 = jnp.maximum(s, 0.0) + x0_blk[...]
        x116[rows, :] = x1.astype(jnp.bfloat16)

    pltpu.emit_pipeline(
        p2_src1, grid=(_NS,),
        in_specs=[pl.BlockSpec((_SRC_BLK, _H), lambda i: (i, 0)),
                  pl.BlockSpec((_SRC_BLK, _D), lambda i: (i, 0))],
    )(hgs_hbm, x0_hbm)

    tar_pipeline(hgt_hbm)

    def p4_src2(x0_blk, out_blk):
        i = pl.program_id(0)
        rows = pl.ds(i * _OUT_BLK, _OUT_BLK)
        s = jnp.dot(hgs16[rows, :], t32[...].astype(jnp.bfloat16),
                    preferred_element_type=jnp.float32)
        out_blk[...] = (w0 * x0_blk[...]
                        + (w1 + w2) * x116[rows, :].astype(jnp.float32)
                        + w2 * jnp.maximum(s, 0.0))

    pltpu.emit_pipeline(
        p4_src2, grid=(_NO,),
        in_specs=[pl.BlockSpec((_OUT_BLK, _D), lambda i: (i, 0))],
        out_specs=[pl.BlockSpec((_OUT_BLK, _D), lambda i: (i, 0))],
    )(x0_hbm, out_hbm)


def kernel(pois_embs, HG_poi_src, HG_poi_tar, layer_attention):
    att2d = layer_attention.reshape(1, -1)
    return pl.pallas_call(
        _mega_kernel,
        in_specs=[
            pl.BlockSpec(memory_space=pl.ANY),
            pl.BlockSpec(memory_space=pl.ANY),
            pl.BlockSpec(memory_space=pl.ANY),
            pl.BlockSpec((1, 3), lambda: (0, 0)),
        ],
        out_specs=pl.BlockSpec(memory_space=pl.ANY),
        out_shape=jax.ShapeDtypeStruct((_N, _D), jnp.float32),
        scratch_shapes=[
            pltpu.VMEM((_N, _H), jnp.bfloat16),
            pltpu.VMEM((_NP, _D), jnp.bfloat16),
            pltpu.VMEM((_H, _D), jnp.float32),
            pltpu.SemaphoreType.DMA,
        ],
    )(HG_poi_tar, HG_poi_src, pois_embs, att2d)


# speedup vs baseline: 1.0513x; 1.0513x over previous
import jax
import jax.numpy as jnp
from jax.experimental import pallas as pl
from jax.experimental.pallas import tpu as pltpu

_N = 10000
_H = 2048
_D = 128

_TAR_BLK = 128
_SRC_BLK = 400
_NT = _H // _TAR_BLK
_NS = _N // _SRC_BLK


def _mega_kernel(hgt_hbm, hgs_hbm, x0_hbm, att_ref, out_hbm):
    pl.run_scoped(
        lambda hgs16, x0f, x116, t16, sem: _mega_body(
            hgt_hbm, hgs_hbm, x0_hbm, att_ref, out_hbm,
            hgs16, x0f, x116, t16, sem),
        pltpu.VMEM((_N, _H), jnp.bfloat16),
        pltpu.VMEM((_N, _D), jnp.float32),
        pltpu.VMEM((_N, _D), jnp.bfloat16),
        pltpu.VMEM((_H, _D), jnp.bfloat16),
        pltpu.SemaphoreType.DMA,
    )


def _mega_body(hgt_hbm, hgs_hbm, x0_hbm, att_ref, out_hbm,
               hgs16, x0f, x116, t16, sem):
    cp = pltpu.make_async_copy(x0_hbm, x0f, sem)
    cp.start()
    cp.wait()
    x116[...] = x0f[...].astype(jnp.bfloat16)

    a = att_ref[0, :]
    e = jnp.exp(a - jnp.max(a))
    w = e / jnp.sum(e)
    w0, w1, w2 = w[0], w[1], w[2]

    def p1_tar1(hgt_blk):
        i = pl.program_id(0)
        blk = hgt_blk[...].astype(jnp.bfloat16)
        r = jnp.dot(blk, x116[...], preferred_element_type=jnp.float32)
        t16[pl.ds(i * _TAR_BLK, _TAR_BLK), :] = r.astype(jnp.bfloat16)

    pltpu.emit_pipeline(
        p1_tar1, grid=(_NT,),
        in_specs=[pl.BlockSpec((_TAR_BLK, _N), lambda i: (i, 0))],
    )(hgt_hbm)

    def p2_src1(hgs_blk):
        i = pl.program_id(0)
        rows = pl.ds(i * _SRC_BLK, _SRC_BLK)
        blk = hgs_blk[...].astype(jnp.bfloat16)
        hgs16[rows, :] = blk
        s = jnp.dot(blk, t16[...], preferred_element_type=jnp.float32)
        x116[rows, :] = (jnp.maximum(s, 0.0) + x0f[rows, :]).astype(jnp.bfloat16)

    pltpu.emit_pipeline(
        p2_src1, grid=(_NS,),
        in_specs=[pl.BlockSpec((_SRC_BLK, _H), lambda i: (i, 0))],
    )(hgs_hbm)

    def p3_tar2(hgt_blk):
        i = pl.program_id(0)
        blk = hgt_blk[...].astype(jnp.bfloat16)
        r = jnp.dot(blk, x116[...], preferred_element_type=jnp.float32)
        t16[pl.ds(i * _TAR_BLK, _TAR_BLK), :] = r.astype(jnp.bfloat16)

    pltpu.emit_pipeline(
        p3_tar2, grid=(_NT,),
        in_specs=[pl.BlockSpec((_TAR_BLK, _N), lambda i: (i, 0))],
    )(hgt_hbm)

    def p4_src2(out_blk):
        i = pl.program_id(0)
        rows = pl.ds(i * _SRC_BLK, _SRC_BLK)
        s = jnp.dot(hgs16[rows, :], t16[...],
                    preferred_element_type=jnp.float32)
        out_blk[...] = (w0 * x0f[rows, :]
                        + (w1 + w2) * x116[rows, :].astype(jnp.float32)
                        + w2 * jnp.maximum(s, 0.0))

    pltpu.emit_pipeline(
        p4_src2, grid=(_NS,),
        out_specs=[pl.BlockSpec((_SRC_BLK, _D), lambda i: (i, 0))],
    )(out_hbm)


def kernel(pois_embs, HG_poi_src, HG_poi_tar, layer_attention):
    att2d = layer_attention.reshape(1, -1)
    return pl.pallas_call(
        _mega_kernel,
        in_specs=[
            pl.BlockSpec(memory_space=pl.ANY),
            pl.BlockSpec(memory_space=pl.ANY),
            pl.BlockSpec(memory_space=pl.ANY),
            pl.BlockSpec((1, 3), lambda: (0, 0)),
        ],
        out_specs=pl.BlockSpec(memory_space=pl.ANY),
        out_shape=jax.ShapeDtypeStruct((_N, _D), jnp.float32),
    )(HG_poi_tar, HG_poi_src, pois_embs, att2d)
